# baseline (speedup 1.0000x reference)
import functools

import jax
import jax.numpy as jnp
from jax import lax
from jax.experimental import pallas as pl
from jax.experimental.pallas import tpu as pltpu
from jax.experimental.pallas import tpu_sc as plsc

N = 10000
H = 128
E = 320000
NPAD = 10240
NC = 2
NS = 16
NW = NC * NS
EPT = 10240
E2 = NW * EPT

IW = 128
IRPT = EPT // IW
CH = 5
EC = E2 // CH
EPTC = EPT // CH
IRC = EPTC // IW
NR = EC // IW
CG = 256
GG = EPT // CG
CS = 256
GS = EPT // CS
RPT = NPAD // NS

_MESH = plsc.VectorSubcoreMesh(
    core_axis_name="c", subcore_axis_name="s", num_cores=NC, num_subcores=NS)


def _pre_ab_body(h_ref, wt_ref, wb_ref, be1_ref, a_ref, b_ref):
    hb = h_ref[...]
    a_ref[...] = jnp.dot(hb, wt_ref[...], preferred_element_type=jnp.float32) + be1_ref[...]
    b_ref[...] = jnp.dot(hb, wb_ref[...], preferred_element_type=jnp.float32)


def _pre_ab(hp, wt, wb, be1):
    bn = 512
    return pl.pallas_call(
        _pre_ab_body,
        grid=(NPAD // bn,),
        in_specs=[
            pl.BlockSpec((bn, H), lambda i: (i, 0)),
            pl.BlockSpec((H, H), lambda i: (0, 0)),
            pl.BlockSpec((H, H), lambda i: (0, 0)),
            pl.BlockSpec((1, H), lambda i: (0, 0)),
        ],
        out_specs=[
            pl.BlockSpec((bn, H), lambda i: (i, 0)),
            pl.BlockSpec((bn, H), lambda i: (i, 0)),
        ],
        out_shape=[
            jax.ShapeDtypeStruct((NPAD, H), jnp.float32),
            jax.ShapeDtypeStruct((NPAD, H), jnp.float32),
        ],
    )(hp, wt, wb, be1)


@functools.partial(
    pl.kernel,
    out_type=jax.ShapeDtypeStruct((E2,), jnp.float32),
    mesh=_MESH,
    scratch_types=[
        pltpu.VMEM((NPAD,), jnp.float32),
        pltpu.VMEM((NPAD,), jnp.float32),
        pltpu.VMEM((NPAD,), jnp.float32),
        pltpu.VMEM((EPT,), jnp.int32),
        pltpu.VMEM((EPT,), jnp.int32),
        pltpu.VMEM((EPT,), jnp.float32),
    ],
    compiler_params=pltpu.CompilerParams(needs_layout_passes=False),
)
def _dsq_kernel(x0_hbm, x1_hbm, x2_hbm, dst_hbm, src_hbm, dsq_hbm,
                x0v, x1v, x2v, dstv, srcv, dsqv):
    wid = lax.axis_index("s") * NC + lax.axis_index("c")
    base = wid * EPT
    pltpu.sync_copy(x0_hbm, x0v)
    pltpu.sync_copy(x1_hbm, x1v)
    pltpu.sync_copy(x2_hbm, x2v)
    pltpu.sync_copy(dst_hbm.at[pl.ds(base, EPT)], dstv)
    pltpu.sync_copy(src_hbm.at[pl.ds(base, EPT)], srcv)

    def body(t, carry):
        o = t * 16
        di = dstv[pl.ds(o, 16)]
        si = srcv[pl.ds(o, 16)]
        d0 = plsc.load_gather(x0v, [di]) - plsc.load_gather(x0v, [si])
        d1 = plsc.load_gather(x1v, [di]) - plsc.load_gather(x1v, [si])
        d2 = plsc.load_gather(x2v, [di]) - plsc.load_gather(x2v, [si])
        dsqv[pl.ds(o, 16)] = d0 * d0 + d1 * d1 + d2 * d2
        return carry

    lax.fori_loop(0, EPT // 16, body, 0)
    pltpu.sync_copy(dsqv, dsq_hbm.at[pl.ds(base, EPT)])


GP = IRC
NSL = 4


@functools.partial(
    pl.kernel,
    out_type=jax.ShapeDtypeStruct((EC, H), jnp.float32),
    mesh=_MESH,
    scratch_types=[
        pltpu.VMEM((IRC, IW), jnp.int32),
        pltpu.VMEM((IRC, IW), jnp.int32),
        pltpu.VMEM((IW, H), jnp.float32),
        pltpu.VMEM((IW, H), jnp.float32),
        pltpu.VMEM((IW, H), jnp.float32),
        pltpu.VMEM((IW, H), jnp.float32),
        pltpu.SemaphoreType.DMA,
        pltpu.SemaphoreType.DMA,
        pltpu.SemaphoreType.DMA,
        pltpu.SemaphoreType.DMA,
        pltpu.SemaphoreType.DMA,
        pltpu.SemaphoreType.DMA,
        pltpu.SemaphoreType.DMA,
        pltpu.SemaphoreType.DMA,
        pltpu.SemaphoreType.DMA,
        pltpu.SemaphoreType.DMA,
        pltpu.SemaphoreType.DMA,
        pltpu.SemaphoreType.DMA,
    ],
)
def _gather_kernel(a_hbm, b_hbm, dst2_hbm, src2_hbm, z_hbm,
                   dstv, srcv, v0, v1, v2, v3,
                   ga0, ga1, ga2, ga3, gb0, gb1, gb2, gb3,
                   ss0, ss1, ss2, ss3):
    wid = lax.axis_index("s") * NC + lax.axis_index("c")
    pltpu.sync_copy(dst2_hbm.at[pl.ds(wid * IRC, IRC)], dstv)
    pltpu.sync_copy(src2_hbm.at[pl.ds(wid * IRC, IRC)], srcv)
    bufs = (v0, v1, v2, v3)
    gas = (ga0, ga1, ga2, ga3)
    gbs = (gb0, gb1, gb2, gb3)
    sss = (ss0, ss1, ss2, ss3)

    NSPL = 4

    def issue_a(g, s):
        for q in range(NSPL):
            qs = pl.ds(q * (IW // NSPL), IW // NSPL)
            pltpu.async_copy(a_hbm.at[dstv.at[g, qs]], bufs[s].at[qs], gas[s])

    def wait_a(s):
        for q in range(NSPL):
            qs = pl.ds(q * (IW // NSPL), IW // NSPL)
            pltpu.make_async_copy(
                a_hbm.at[dstv.at[0, qs]], bufs[s].at[qs], gas[s]).wait()

    def issue_b(g, s):
        for q in range(NSPL):
            qs = pl.ds(q * (IW // NSPL), IW // NSPL)
            pltpu.async_copy(b_hbm.at[srcv.at[g, qs]], bufs[s].at[qs],
                             gbs[s], add=True)

    def wait_b(s):
        for q in range(NSPL):
            qs = pl.ds(q * (IW // NSPL), IW // NSPL)
            pltpu.make_async_copy(
                b_hbm.at[srcv.at[0, qs]], bufs[s].at[qs], gbs[s]).wait()

    def issue_store(g, s):
        pltpu.async_copy(bufs[s], z_hbm.at[pl.ds(wid * EPTC + g * IW, IW)],
                         sss[s])

    def wait_store(s):
        pltpu.make_async_copy(bufs[s], z_hbm.at[pl.ds(0, IW)], sss[s]).wait()

    for s in range(NSL):
        issue_a(s, s)
    for s in range(NSL):
        wait_a(s)
        issue_b(s, s)

    def quad(k, carry):
        g = 4 * k
        for j in range(NSL):
            wait_b(j)
            issue_store(g + j, j)
        for j in range(NSL):
            wait_store(j)
            issue_a(g + 4 + j, j)
        for j in range(NSL):
            wait_a(j)
            issue_b(g + 4 + j, j)
        return carry

    lax.fori_loop(0, GP // 4 - 1, quad, 0)

    for j in range(NSL):
        wait_b(j)
        issue_store(GP - 4 + j, j)
    for j in range(NSL):
        wait_store(j)


def _edge_mlp_body(z_ref, dsq_ref, w2_ref, b2_ref, winf_ref, binf_ref, w_ref):
    z = jnp.maximum(z_ref[...], 0.0).astype(jnp.bfloat16)
    w2 = w2_ref[...].astype(jnp.bfloat16)
    m = jnp.maximum(
        jnp.dot(z, w2, preferred_element_type=jnp.float32) + b2_ref[...], 0.0)
    eij = jnp.sum(m * winf_ref[...], axis=1, keepdims=True) + binf_ref[...]
    dsq = dsq_ref[...]
    edge_dis = jax.nn.sigmoid(30.0 / (jnp.sqrt(dsq) + 1e-08))
    ew = jax.nn.sigmoid(eij * edge_dis)
    w_ref[...] = m * ew


def _edge_mlp(z, dsq1, w2, b2, winf, binf):
    be = 2048
    return pl.pallas_call(
        _edge_mlp_body,
        grid=(EC // be,),
        in_specs=[
            pl.BlockSpec((be, H), lambda i: (i, 0)),
            pl.BlockSpec((be, 1), lambda i: (i, 0)),
            pl.BlockSpec((H, H), lambda i: (0, 0)),
            pl.BlockSpec((1, H), lambda i: (0, 0)),
            pl.BlockSpec((1, H), lambda i: (0, 0)),
            pl.BlockSpec((1, 1), lambda i: (0, 0)),
        ],
        out_specs=pl.BlockSpec((be, H), lambda i: (i, 0)),
        out_shape=jax.ShapeDtypeStruct((EC, H), jnp.float32),
    )(z, dsq1, w2, b2, winf, binf)


@functools.partial(
    pl.kernel,
    out_type=jax.ShapeDtypeStruct((NC, NPAD, H), jnp.float32),
    mesh=_MESH,
    scratch_types=[
        pltpu.VMEM((IRPT, IW), jnp.int32),
        pltpu.VMEM((IW, H), jnp.float32),
        pltpu.VMEM((IW, H), jnp.float32),
        pltpu.VMEM_SHARED((NPAD, H), jnp.float32),
        pltpu.SemaphoreType.DMA,
        pltpu.SemaphoreType.DMA,
        pltpu.SemaphoreType.DMA,
        pltpu.SemaphoreType.DMA,
    ],
)
def _scatter_kernel(w0_hbm, w1_hbm, w2_hbm, w3_hbm, w4_hbm, dst2_hbm,
                    out_hbm, dstv, wv0, wv1, acc, ls0, ls1, cs0, cs1):
    c = lax.axis_index("c")
    s = lax.axis_index("s")
    wid = s * NC + c
    myrow = s * RPT
    ws = (w0_hbm, w1_hbm, w2_hbm, w3_hbm, w4_hbm)

    for k in range(CH):
        pltpu.sync_copy(dst2_hbm.at[pl.ds(k * NR + wid * IRC, IRC)],
                        dstv.at[pl.ds(k * IRC, IRC)])

    def zrow(i, carry):
        for j in range(H // 16):
            wv0[i, pl.ds(j * 16, 16)] = jnp.zeros((16,), jnp.float32)
        return carry

    lax.fori_loop(0, IW, zrow, 0)
    for t in range(RPT // IW):
        pltpu.sync_copy(wv0.at[pl.ds(0, IW)],
                        acc.at[pl.ds(myrow + t * IW, IW)])
    plsc.subcore_barrier()

    wvs = (wv0, wv1)
    lss = (ls0, ls1)
    css = (cs0, cs1)

    NG = CH * IRC

    def issue_load(g, s_):
        k, loc = divmod(g, IRC)
        pltpu.async_copy(ws[k].at[pl.ds(wid * EPTC + loc * IW, IW)],
                         wvs[s_], lss[s_])

    def wait_load(s_):
        pltpu.make_async_copy(w0_hbm.at[pl.ds(0, IW)], wvs[s_],
                              lss[s_]).wait()

    def issue_scat(g, s_):
        pltpu.async_copy(wvs[s_], acc.at[dstv.at[g]], css[s_], add=True)

    def wait_scat(s_):
        pltpu.make_async_copy(wvs[s_], acc.at[dstv.at[0]], css[s_]).wait()

    issue_load(0, 0)
    wait_load(0)
    issue_load(1, 1)
    issue_scat(0, 0)
    for g in range(1, NG):
        s_ = g & 1
        wait_load(s_)
        wait_scat(1 - s_)
        if g + 1 < NG:
            issue_load(g + 1, 1 - s_)
        issue_scat(g, s_)
    wait_scat((NG - 1) & 1)
    plsc.subcore_barrier()

    for t in range(RPT // IW):
        pltpu.sync_copy(acc.at[pl.ds(myrow + t * IW, IW)],
                        wv0.at[pl.ds(0, IW)])
        pltpu.sync_copy(wv0.at[pl.ds(0, IW)],
                        out_hbm.at[c].at[pl.ds(myrow + t * IW, IW)])


def _node_mlp_body(p0_ref, p1_ref, h_ref, wt_ref, wb_ref, bn1_ref, w2_ref,
                   bn2_ref, g_ref, b_ref, out_ref):
    mi = p0_ref[...] + p1_ref[...]
    hb = h_ref[...]
    t = jnp.maximum(
        jnp.dot(mi, wt_ref[...], preferred_element_type=jnp.float32)
        + jnp.dot(hb, wb_ref[...], preferred_element_type=jnp.float32)
        + bn1_ref[...], 0.0)
    upd = jnp.dot(t, w2_ref[...], preferred_element_type=jnp.float32) + bn2_ref[...]
    hn = hb + upd
    mu = jnp.mean(hn, axis=-1, keepdims=True)
    var = jnp.mean((hn - mu) ** 2, axis=-1, keepdims=True)
    out_ref[...] = (hn - mu) * lax.rsqrt(var + 1e-05) * g_ref[...] + b_ref[...]


def _node_mlp(p0, p1, hp, wt, wb, bn1, w2, bn2, g, b):
    bn = 512
    return pl.pallas_call(
        _node_mlp_body,
        grid=(NPAD // bn,),
        in_specs=[
            pl.BlockSpec((bn, H), lambda i: (i, 0)),
            pl.BlockSpec((bn, H), lambda i: (i, 0)),
            pl.BlockSpec((bn, H), lambda i: (i, 0)),
            pl.BlockSpec((H, H), lambda i: (0, 0)),
            pl.BlockSpec((H, H), lambda i: (0, 0)),
            pl.BlockSpec((1, H), lambda i: (0, 0)),
            pl.BlockSpec((H, H), lambda i: (0, 0)),
            pl.BlockSpec((1, H), lambda i: (0, 0)),
            pl.BlockSpec((1, H), lambda i: (0, 0)),
            pl.BlockSpec((1, H), lambda i: (0, 0)),
        ],
        out_specs=pl.BlockSpec((bn, H), lambda i: (i, 0)),
        out_shape=jax.ShapeDtypeStruct((NPAD, H), jnp.float32),
    )(p0, p1, hp, wt, wb, bn1, w2, bn2, g, b)


def kernel(h, x, edge_index, We1, be1, We2, be2, Winf, binf, Wn1, bn1, Wn2,
           bn2, ln_g, ln_b):
    pad_id = N + jnp.arange(E2 - E, dtype=jnp.int32) % (NPAD - N)
    src = jnp.concatenate([edge_index[0], pad_id])
    dst = jnp.concatenate([edge_index[1], pad_id])
    src_r = src.reshape(NW, CH, EPTC).transpose(1, 0, 2).reshape(E2)
    dst_r = dst.reshape(NW, CH, EPTC).transpose(1, 0, 2).reshape(E2)
    dst2 = dst_r.reshape(E2 // IW, IW)
    src2 = src_r.reshape(E2 // IW, IW)

    hp = jnp.pad(h, ((0, NPAD - N), (0, 0)))
    a, b = _pre_ab(hp, We1[:H], We1[H:], be1.reshape(1, H))

    xp = jnp.pad(x, ((0, NPAD - N), (0, 0)))
    dsq = _dsq_kernel(xp[:, 0], xp[:, 1], xp[:, 2], dst_r, src_r)

    ws = []
    for k in range(CH):
        zk = _gather_kernel(a, b, dst2[k * NR:(k + 1) * NR],
                            src2[k * NR:(k + 1) * NR])
        ws.append(_edge_mlp(zk, lax.slice(dsq, (k * EC,), ((k + 1) * EC,))
                            .reshape(EC, 1), We2, be2.reshape(1, H),
                            Winf.reshape(1, H), binf.reshape(1, 1)))

    parts = _scatter_kernel(ws[0], ws[1], ws[2], ws[3], ws[4], dst2)

    hn = _node_mlp(parts[0], parts[1], hp, Wn1[:H], Wn1[H:],
                   bn1.reshape(1, H), Wn2, bn2.reshape(1, H),
                   ln_g.reshape(1, H), ln_b.reshape(1, H))
    return (hn[:N], x)


# speedup vs baseline: 1.0189x; 1.0189x over previous
import functools

import jax
import jax.numpy as jnp
from jax import lax
from jax.experimental import pallas as pl
from jax.experimental.pallas import tpu as pltpu
from jax.experimental.pallas import tpu_sc as plsc

N = 10000
H = 128
E = 320000
NPAD = 10240
NC = 2
NS = 16
NW = NC * NS
EPT = 10240
E2 = NW * EPT

IW = 128
IRPT = EPT // IW
CH = 5
EC = E2 // CH
EPTC = EPT // CH
IRC = EPTC // IW
NR = EC // IW
CG = 256
GG = EPT // CG
CS = 256
GS = EPT // CS
RPT = NPAD // NS

_MESH = plsc.VectorSubcoreMesh(
    core_axis_name="c", subcore_axis_name="s", num_cores=NC, num_subcores=NS)


def _pre_ab_body(h_ref, wt_ref, wb_ref, be1_ref, a_ref, b_ref):
    hb = h_ref[...]
    a_ref[...] = jnp.dot(hb, wt_ref[...], preferred_element_type=jnp.float32) + be1_ref[...]
    b_ref[...] = jnp.dot(hb, wb_ref[...], preferred_element_type=jnp.float32)


def _pre_ab(hp, wt, wb, be1):
    bn = 512
    return pl.pallas_call(
        _pre_ab_body,
        grid=(NPAD // bn,),
        in_specs=[
            pl.BlockSpec((bn, H), lambda i: (i, 0)),
            pl.BlockSpec((H, H), lambda i: (0, 0)),
            pl.BlockSpec((H, H), lambda i: (0, 0)),
            pl.BlockSpec((1, H), lambda i: (0, 0)),
        ],
        out_specs=[
            pl.BlockSpec((bn, H), lambda i: (i, 0)),
            pl.BlockSpec((bn, H), lambda i: (i, 0)),
        ],
        out_shape=[
            jax.ShapeDtypeStruct((NPAD, H), jnp.float32),
            jax.ShapeDtypeStruct((NPAD, H), jnp.float32),
        ],
    )(hp, wt, wb, be1)


@functools.partial(
    pl.kernel,
    out_type=jax.ShapeDtypeStruct((E2,), jnp.float32),
    mesh=_MESH,
    scratch_types=[
        pltpu.VMEM((NPAD,), jnp.float32),
        pltpu.VMEM((NPAD,), jnp.float32),
        pltpu.VMEM((NPAD,), jnp.float32),
        pltpu.VMEM((EPT,), jnp.int32),
        pltpu.VMEM((EPT,), jnp.int32),
        pltpu.VMEM((EPT,), jnp.float32),
    ],
    compiler_params=pltpu.CompilerParams(needs_layout_passes=False),
)
def _dsq_kernel(x0_hbm, x1_hbm, x2_hbm, dst_hbm, src_hbm, dsq_hbm,
                x0v, x1v, x2v, dstv, srcv, dsqv):
    wid = lax.axis_index("s") * NC + lax.axis_index("c")
    base = wid * EPT
    pltpu.sync_copy(x0_hbm, x0v)
    pltpu.sync_copy(x1_hbm, x1v)
    pltpu.sync_copy(x2_hbm, x2v)
    pltpu.sync_copy(dst_hbm.at[pl.ds(base, EPT)], dstv)
    pltpu.sync_copy(src_hbm.at[pl.ds(base, EPT)], srcv)

    def body(t, carry):
        o = t * 16
        di = dstv[pl.ds(o, 16)]
        si = srcv[pl.ds(o, 16)]
        d0 = plsc.load_gather(x0v, [di]) - plsc.load_gather(x0v, [si])
        d1 = plsc.load_gather(x1v, [di]) - plsc.load_gather(x1v, [si])
        d2 = plsc.load_gather(x2v, [di]) - plsc.load_gather(x2v, [si])
        dsqv[pl.ds(o, 16)] = d0 * d0 + d1 * d1 + d2 * d2
        return carry

    lax.fori_loop(0, EPT // 16, body, 0)
    pltpu.sync_copy(dsqv, dsq_hbm.at[pl.ds(base, EPT)])


GP = IRC
NSL = 4


@functools.partial(
    pl.kernel,
    out_type=jax.ShapeDtypeStruct((EC, H), jnp.float32),
    mesh=_MESH,
    scratch_types=[
        pltpu.VMEM((IRC, IW), jnp.int32),
        pltpu.VMEM((IRC, IW), jnp.int32),
        pltpu.VMEM((IW, H), jnp.float32),
        pltpu.VMEM((IW, H), jnp.float32),
        pltpu.VMEM((IW, H), jnp.float32),
        pltpu.VMEM((IW, H), jnp.float32),
        pltpu.SemaphoreType.DMA,
        pltpu.SemaphoreType.DMA,
        pltpu.SemaphoreType.DMA,
        pltpu.SemaphoreType.DMA,
        pltpu.SemaphoreType.DMA,
        pltpu.SemaphoreType.DMA,
        pltpu.SemaphoreType.DMA,
        pltpu.SemaphoreType.DMA,
        pltpu.SemaphoreType.DMA,
        pltpu.SemaphoreType.DMA,
        pltpu.SemaphoreType.DMA,
        pltpu.SemaphoreType.DMA,
    ],
)
def _gather_kernel(a_hbm, b_hbm, dst2_hbm, src2_hbm, z_hbm,
                   dstv, srcv, v0, v1, v2, v3,
                   ga0, ga1, ga2, ga3, gb0, gb1, gb2, gb3,
                   ss0, ss1, ss2, ss3):
    wid = lax.axis_index("s") * NC + lax.axis_index("c")
    pltpu.sync_copy(dst2_hbm.at[pl.ds(wid * IRC, IRC)], dstv)
    pltpu.sync_copy(src2_hbm.at[pl.ds(wid * IRC, IRC)], srcv)
    bufs = (v0, v1, v2, v3)
    gas = (ga0, ga1, ga2, ga3)
    gbs = (gb0, gb1, gb2, gb3)
    sss = (ss0, ss1, ss2, ss3)

    NSPL = 4

    def issue_a(g, s):
        for q in range(NSPL):
            qs = pl.ds(q * (IW // NSPL), IW // NSPL)
            pltpu.async_copy(a_hbm.at[dstv.at[g, qs]], bufs[s].at[qs], gas[s])

    def wait_a(s):
        for q in range(NSPL):
            qs = pl.ds(q * (IW // NSPL), IW // NSPL)
            pltpu.make_async_copy(
                a_hbm.at[dstv.at[0, qs]], bufs[s].at[qs], gas[s]).wait()

    def issue_b(g, s):
        for q in range(NSPL):
            qs = pl.ds(q * (IW // NSPL), IW // NSPL)
            pltpu.async_copy(b_hbm.at[srcv.at[g, qs]], bufs[s].at[qs],
                             gbs[s], add=True)

    def wait_b(s):
        for q in range(NSPL):
            qs = pl.ds(q * (IW // NSPL), IW // NSPL)
            pltpu.make_async_copy(
                b_hbm.at[srcv.at[0, qs]], bufs[s].at[qs], gbs[s]).wait()

    def issue_store(g, s):
        pltpu.async_copy(bufs[s], z_hbm.at[pl.ds(wid * EPTC + g * IW, IW)],
                         sss[s])

    def wait_store(s):
        pltpu.make_async_copy(bufs[s], z_hbm.at[pl.ds(0, IW)], sss[s]).wait()

    for s in range(NSL):
        issue_a(s, s)
    for s in range(NSL):
        wait_a(s)
        issue_b(s, s)

    def quad(k, carry):
        g = 4 * k
        for j in range(NSL):
            wait_b(j)
            issue_store(g + j, j)
        for j in range(NSL):
            wait_store(j)
            issue_a(g + 4 + j, j)
        for j in range(NSL):
            wait_a(j)
            issue_b(g + 4 + j, j)
        return carry

    lax.fori_loop(0, GP // 4 - 1, quad, 0)

    for j in range(NSL):
        wait_b(j)
        issue_store(GP - 4 + j, j)
    for j in range(NSL):
        wait_store(j)


def _edge_mlp_body(z_ref, dsq_ref, w2_ref, b2_ref, winf_ref, binf_ref, w_ref):
    z = jnp.maximum(z_ref[...], 0.0).astype(jnp.bfloat16)
    w2 = w2_ref[...].astype(jnp.bfloat16)
    m = jnp.maximum(
        jnp.dot(z, w2, preferred_element_type=jnp.float32) + b2_ref[...], 0.0)
    eij = jnp.sum(m * winf_ref[...], axis=1, keepdims=True) + binf_ref[...]
    dsq = dsq_ref[...].astype(jnp.float32)
    edge_dis = jax.nn.sigmoid(30.0 / (jnp.sqrt(dsq) + 1e-08))
    ew = jax.nn.sigmoid(eij * edge_dis)
    w_ref[...] = m * ew


def _edge_mlp(z, dsq1, w2, b2, winf, binf):
    be = 2048
    return pl.pallas_call(
        _edge_mlp_body,
        grid=(EC // be,),
        in_specs=[
            pl.BlockSpec((be, H), lambda i: (i, 0)),
            pl.BlockSpec((be, 1), lambda i: (i, 0)),
            pl.BlockSpec((H, H), lambda i: (0, 0)),
            pl.BlockSpec((1, H), lambda i: (0, 0)),
            pl.BlockSpec((1, H), lambda i: (0, 0)),
            pl.BlockSpec((1, 1), lambda i: (0, 0)),
        ],
        out_specs=pl.BlockSpec((be, H), lambda i: (i, 0)),
        out_shape=jax.ShapeDtypeStruct((EC, H), jnp.float32),
    )(z, dsq1, w2, b2, winf, binf)


@functools.partial(
    pl.kernel,
    out_type=jax.ShapeDtypeStruct((NC, NPAD, H), jnp.float32),
    mesh=_MESH,
    scratch_types=[
        pltpu.VMEM((IRPT, IW), jnp.int32),
        pltpu.VMEM((IW, H), jnp.float32),
        pltpu.VMEM((IW, H), jnp.float32),
        pltpu.VMEM_SHARED((NPAD, H), jnp.float32),
        pltpu.SemaphoreType.DMA,
        pltpu.SemaphoreType.DMA,
        pltpu.SemaphoreType.DMA,
        pltpu.SemaphoreType.DMA,
    ],
)
def _scatter_kernel(w0_hbm, w1_hbm, w2_hbm, w3_hbm, w4_hbm, dst2_hbm,
                    out_hbm, dstv, wv0, wv1, acc, ls0, ls1, cs0, cs1):
    c = lax.axis_index("c")
    s = lax.axis_index("s")
    wid = s * NC + c
    myrow = s * RPT
    ws = (w0_hbm, w1_hbm, w2_hbm, w3_hbm, w4_hbm)

    for k in range(CH):
        pltpu.sync_copy(dst2_hbm.at[pl.ds(k * NR + wid * IRC, IRC)],
                        dstv.at[pl.ds(k * IRC, IRC)])

    def zrow(i, carry):
        for j in range(H // 16):
            wv0[i, pl.ds(j * 16, 16)] = jnp.zeros((16,), jnp.float32)
        return carry

    lax.fori_loop(0, IW, zrow, 0)
    for t in range(RPT // IW):
        pltpu.sync_copy(wv0.at[pl.ds(0, IW)],
                        acc.at[pl.ds(myrow + t * IW, IW)])
    plsc.subcore_barrier()

    wvs = (wv0, wv1)
    lss = (ls0, ls1)
    css = (cs0, cs1)

    NG = CH * IRC

    def issue_load(g, s_):
        k, loc = divmod(g, IRC)
        pltpu.async_copy(ws[k].at[pl.ds(wid * EPTC + loc * IW, IW)],
                         wvs[s_], lss[s_])

    def wait_load(s_):
        pltpu.make_async_copy(w0_hbm.at[pl.ds(0, IW)], wvs[s_],
                              lss[s_]).wait()

    def issue_scat(g, s_):
        pltpu.async_copy(wvs[s_], acc.at[dstv.at[g]], css[s_], add=True)

    def wait_scat(s_):
        pltpu.make_async_copy(wvs[s_], acc.at[dstv.at[0]], css[s_]).wait()

    issue_load(0, 0)
    wait_load(0)
    issue_load(1, 1)
    issue_scat(0, 0)
    for g in range(1, NG):
        s_ = g & 1
        wait_load(s_)
        wait_scat(1 - s_)
        if g + 1 < NG:
            issue_load(g + 1, 1 - s_)
        issue_scat(g, s_)
    wait_scat((NG - 1) & 1)
    plsc.subcore_barrier()

    for t in range(RPT // IW):
        pltpu.sync_copy(acc.at[pl.ds(myrow + t * IW, IW)],
                        wv0.at[pl.ds(0, IW)])
        pltpu.sync_copy(wv0.at[pl.ds(0, IW)],
                        out_hbm.at[c].at[pl.ds(myrow + t * IW, IW)])


def _node_mlp_body(p0_ref, p1_ref, h_ref, wt_ref, wb_ref, bn1_ref, w2_ref,
                   bn2_ref, g_ref, b_ref, out_ref):
    mi = p0_ref[...] + p1_ref[...]
    hb = h_ref[...]
    t = jnp.maximum(
        jnp.dot(mi, wt_ref[...], preferred_element_type=jnp.float32)
        + jnp.dot(hb, wb_ref[...], preferred_element_type=jnp.float32)
        + bn1_ref[...], 0.0)
    upd = jnp.dot(t, w2_ref[...], preferred_element_type=jnp.float32) + bn2_ref[...]
    hn = hb + upd
    mu = jnp.mean(hn, axis=-1, keepdims=True)
    var = jnp.mean((hn - mu) ** 2, axis=-1, keepdims=True)
    out_ref[...] = (hn - mu) * lax.rsqrt(var + 1e-05) * g_ref[...] + b_ref[...]


def _node_mlp(p0, p1, hp, wt, wb, bn1, w2, bn2, g, b):
    bn = 512
    return pl.pallas_call(
        _node_mlp_body,
        grid=(NPAD // bn,),
        in_specs=[
            pl.BlockSpec((bn, H), lambda i: (i, 0)),
            pl.BlockSpec((bn, H), lambda i: (i, 0)),
            pl.BlockSpec((bn, H), lambda i: (i, 0)),
            pl.BlockSpec((H, H), lambda i: (0, 0)),
            pl.BlockSpec((H, H), lambda i: (0, 0)),
            pl.BlockSpec((1, H), lambda i: (0, 0)),
            pl.BlockSpec((H, H), lambda i: (0, 0)),
            pl.BlockSpec((1, H), lambda i: (0, 0)),
            pl.BlockSpec((1, H), lambda i: (0, 0)),
            pl.BlockSpec((1, H), lambda i: (0, 0)),
        ],
        out_specs=pl.BlockSpec((bn, H), lambda i: (i, 0)),
        out_shape=jax.ShapeDtypeStruct((NPAD, H), jnp.float32),
    )(p0, p1, hp, wt, wb, bn1, w2, bn2, g, b)


def kernel(h, x, edge_index, We1, be1, We2, be2, Winf, binf, Wn1, bn1, Wn2,
           bn2, ln_g, ln_b):
    pad_id = N + jnp.arange(E2 - E, dtype=jnp.int32) % (NPAD - N)
    src = jnp.concatenate([edge_index[0], pad_id])
    dst = jnp.concatenate([edge_index[1], pad_id])
    src_r = src.reshape(NW, CH, EPTC).transpose(1, 0, 2).reshape(E2)
    dst_r = dst.reshape(NW, CH, EPTC).transpose(1, 0, 2).reshape(E2)
    dst2 = dst_r.reshape(E2 // IW, IW)
    src2 = src_r.reshape(E2 // IW, IW)

    hp = jnp.pad(h, ((0, NPAD - N), (0, 0)))
    a, b = _pre_ab(hp, We1[:H], We1[H:], be1.reshape(1, H))

    xp = jnp.pad(x, ((0, NPAD - N), (0, 0)))
    dsq = _dsq_kernel(xp[:, 0], xp[:, 1], xp[:, 2], dst_r, src_r)

    ws = []
    for k in range(CH):
        zk = _gather_kernel(a, b, dst2[k * NR:(k + 1) * NR],
                            src2[k * NR:(k + 1) * NR])
        ws.append(_edge_mlp(zk, lax.slice(dsq, (k * EC,), ((k + 1) * EC,))
                            .astype(jnp.bfloat16).reshape(EC, 1), We2,
                            be2.reshape(1, H), Winf.reshape(1, H),
                            binf.reshape(1, 1)))

    parts = _scatter_kernel(ws[0], ws[1], ws[2], ws[3], ws[4], dst2)

    hn = _node_mlp(parts[0], parts[1], hp, Wn1[:H], Wn1[H:],
                   bn1.reshape(1, H), Wn2, bn2.reshape(1, H),
                   ln_g.reshape(1, H), ln_b.reshape(1, H))
    return (hn[:N], x)
